# baseline (speedup 1.0000x reference)
import jax
import jax.numpy as jnp
from jax import lax
from jax.experimental import pallas as pl
from jax.experimental.pallas import tpu as pltpu
from jax.experimental.pallas import tpu_sc as plsc

NUM_RESIDUE_TYPE = 32
PAIR_DIM = 128
N_NODES = 10000
N_EDGES = 320000

NC, NS, L = 2, 16, 16
NW = NC * NS
BPW = N_EDGES // NW
CHUNK = 80
C2 = 2 * CHUNK
NCHUNK = BPW // CHUNK
NCHUNK_ALL = N_EDGES // CHUNK
VECS = CHUNK // L
K = 11
MACRO = NCHUNK // K
TAIL = NCHUNK - MACRO * K


def _body(edge_hbm, residue_hbm, weight_hbm, out_hbm, *scratch):
    ev = scratch[0:K]
    ri = scratch[K:2 * K]
    pair = scratch[2 * K:3 * K]
    rows = scratch[3 * K:4 * K]
    asem = scratch[4 * K:5 * K]
    wsem = scratch[5 * K:6 * K]
    shw = scratch[6 * K]
    shr = scratch[6 * K + 1]

    sid = lax.axis_index("s")
    wid = sid * NC + lax.axis_index("c")
    base = wid * BPW
    lane = lax.iota(jnp.int32, L)
    idx_even = (lane % (L // 2)) * 2
    lo_half = lane < (L // 2)
    _dn = lax.GatherDimensionNumbers(offset_dims=(), collapsed_slice_dims=(0,),
                                     start_index_map=(0,))

    def take16(v, idx):
        return lax.gather(v, idx.reshape(L, 1), _dn, slice_sizes=(1,),
                          mode=lax.GatherScatterMode.PROMISE_IN_BOUNDS)

    @pl.when(sid == 0)
    def _():
        pltpu.sync_copy(weight_hbm, shw)
        pltpu.sync_copy(residue_hbm, shr)

    plsc.subcore_barrier()

    def stage_in(ci, b):
        off = 2 * (base + ci * CHUNK)
        pltpu.async_copy(edge_hbm.at[pl.ds(off, C2)], ev[b], asem[b])

    def fire_residue(b):
        pltpu.make_async_copy(edge_hbm.at[pl.ds(0, C2)], ev[b],
                              asem[b]).wait()
        pltpu.async_copy(shr.at[ev[b]], ri[b], asem[b])

    def fire_weight(b, reuse):
        pltpu.make_async_copy(shr.at[ev[b]], ri[b], asem[b]).wait()

        def vec(j, c):
            v0 = ri[b][pl.ds(j * 2 * L, L)]
            v1 = ri[b][pl.ds(j * 2 * L + L, L)]
            s_lo = take16(v0, idx_even)
            s_hi = take16(v1, idx_even)
            d_lo = take16(v0, idx_even + 1)
            d_hi = take16(v1, idx_even + 1)
            s = jnp.where(lo_half, s_lo, s_hi)
            d = jnp.where(lo_half, d_lo, d_hi)
            pair[b][pl.ds(j * L, L)] = (s << 5) | d
            return c

        lax.fori_loop(0, VECS, vec, 0, unroll=5)

        if reuse is not None:
            @pl.when(reuse)
            def _():
                pltpu.make_async_copy(
                    rows[b], out_hbm.at[pl.ds(base, CHUNK)], wsem[b]).wait()

        pltpu.async_copy(shw.at[pair[b]], rows[b], asem[b])

    def fire_out(ci, b):
        pltpu.make_async_copy(shw.at[pair[b]], rows[b], asem[b]).wait()
        pltpu.async_copy(rows[b], out_hbm.at[pl.ds(base + ci * CHUNK, CHUNK)],
                         wsem[b])

    def macro_body(m, carry):
        for b in range(K):
            stage_in(m * K + b, b)
        for b in range(K):
            fire_residue(b)
        for b in range(K):
            fire_weight(b, m > 0)
        for b in range(K):
            fire_out(m * K + b, b)
        return carry

    lax.fori_loop(0, MACRO, macro_body, 0)

    for t in range(TAIL):
        ci = MACRO * K + t
        stage_in(ci, t)
        fire_residue(t)
        fire_weight(t, jnp.bool_(True))
        fire_out(ci, t)

    for b in range(K):
        pltpu.make_async_copy(
            rows[b], out_hbm.at[pl.ds(base, CHUNK)], wsem[b]).wait()


@jax.jit
def kernel(residue, edge_index, weight):
    edge_flat = edge_index.reshape(-1).astype(jnp.int32)
    mesh = plsc.VectorSubcoreMesh(core_axis_name="c", subcore_axis_name="s",
                                  num_cores=NC, num_subcores=NS)
    scratch = (
        [pltpu.VMEM((C2,), jnp.int32) for _ in range(2 * K)]
        + [pltpu.VMEM((CHUNK,), jnp.int32) for _ in range(K)]
        + [pltpu.VMEM((CHUNK, PAIR_DIM), jnp.float32) for _ in range(K)]
        + [pltpu.SemaphoreType.DMA for _ in range(2 * K)]
        + [pltpu.VMEM_SHARED((NUM_RESIDUE_TYPE * NUM_RESIDUE_TYPE, PAIR_DIM),
                             jnp.float32),
           pltpu.VMEM_SHARED((N_NODES,), jnp.int32)]
    )
    fn = pl.kernel(
        _body,
        out_type=jax.ShapeDtypeStruct((N_EDGES, PAIR_DIM), jnp.float32),
        mesh=mesh,
        scratch_types=scratch,
    )
    return fn(edge_flat, residue.astype(jnp.int32), weight)


# speedup vs baseline: 2.4709x; 2.4709x over previous
import jax
import jax.numpy as jnp
from jax import lax
from jax.experimental import pallas as pl
from jax.experimental.pallas import tpu as pltpu
from jax.experimental.pallas import tpu_sc as plsc

NUM_RESIDUE_TYPE = 32
PAIR_DIM = 128
N_NODES = 10000
N_EDGES = 320000

NC, NS, L = 2, 16, 16
NW = NC * NS
BPW = N_EDGES // NW
CHUNK = 80
NCHUNK = BPW // CHUNK
VECS = CHUNK // L
K = 11
MACRO = NCHUNK // K
TAIL = NCHUNK - MACRO * K


def _body(src_hbm, dst_hbm, residue_hbm, weight_hbm, out_hbm, *scratch):
    sv = scratch[0:K]
    dv = scratch[K:2 * K]
    rs = scratch[2 * K:3 * K]
    rd = scratch[3 * K:4 * K]
    pair = scratch[4 * K:5 * K]
    rows = scratch[5 * K:6 * K]
    asem = scratch[6 * K:7 * K]
    wsem = scratch[7 * K:8 * K]
    shw = scratch[8 * K]
    shr = scratch[8 * K + 1]

    sid = lax.axis_index("s")
    wid = sid * NC + lax.axis_index("c")
    base = wid * BPW

    @pl.when(sid == 0)
    def _():
        pltpu.sync_copy(weight_hbm, shw)
        pltpu.sync_copy(residue_hbm, shr)

    plsc.subcore_barrier()

    def stage_in(ci, b):
        off = base + ci * CHUNK
        pltpu.async_copy(src_hbm.at[pl.ds(off, CHUNK)], sv[b], asem[b])
        pltpu.async_copy(dst_hbm.at[pl.ds(off, CHUNK)], dv[b], asem[b])

    def fire_residue(b):
        pltpu.make_async_copy(src_hbm.at[pl.ds(0, CHUNK)], sv[b],
                              asem[b]).wait()
        pltpu.make_async_copy(dst_hbm.at[pl.ds(0, CHUNK)], dv[b],
                              asem[b]).wait()
        pltpu.async_copy(shr.at[sv[b]], rs[b], asem[b])
        pltpu.async_copy(shr.at[dv[b]], rd[b], asem[b])

    def fire_weight(b, reuse):
        pltpu.make_async_copy(shr.at[sv[b]], rs[b], asem[b]).wait()
        pltpu.make_async_copy(shr.at[dv[b]], rd[b], asem[b]).wait()

        def vec(j, c):
            o = j * L
            pair[b][pl.ds(o, L)] = (
                rs[b][pl.ds(o, L)] * NUM_RESIDUE_TYPE + rd[b][pl.ds(o, L)])
            return c

        lax.fori_loop(0, VECS, vec, 0, unroll=5)

        if reuse is not None:
            @pl.when(reuse)
            def _():
                pltpu.make_async_copy(
                    rows[b], out_hbm.at[pl.ds(base, CHUNK)], wsem[b]).wait()

        pltpu.async_copy(shw.at[pair[b]], rows[b], asem[b])

    def fire_out(ci, b):
        pltpu.make_async_copy(shw.at[pair[b]], rows[b], asem[b]).wait()
        pltpu.async_copy(rows[b], out_hbm.at[pl.ds(base + ci * CHUNK, CHUNK)],
                         wsem[b])

    def macro_body(m, carry):
        for b in range(K):
            stage_in(m * K + b, b)
        for b in range(K):
            fire_residue(b)
        for b in range(K):
            fire_weight(b, m > 0)
        for b in range(K):
            fire_out(m * K + b, b)
        return carry

    lax.fori_loop(0, MACRO, macro_body, 0)

    for t in range(TAIL):
        ci = MACRO * K + t
        stage_in(ci, t)
        fire_residue(t)
        fire_weight(t, jnp.bool_(True))
        fire_out(ci, t)

    for b in range(K):
        pltpu.make_async_copy(
            rows[b], out_hbm.at[pl.ds(base, CHUNK)], wsem[b]).wait()


@jax.jit
def kernel(residue, edge_index, weight):
    src = edge_index[:, 0].astype(jnp.int32)
    dst = edge_index[:, 1].astype(jnp.int32)
    mesh = plsc.VectorSubcoreMesh(core_axis_name="c", subcore_axis_name="s",
                                  num_cores=NC, num_subcores=NS)
    scratch = (
        [pltpu.VMEM((CHUNK,), jnp.int32) for _ in range(5 * K)]
        + [pltpu.VMEM((CHUNK, PAIR_DIM), jnp.float32) for _ in range(K)]
        + [pltpu.SemaphoreType.DMA for _ in range(2 * K)]
        + [pltpu.VMEM_SHARED((NUM_RESIDUE_TYPE * NUM_RESIDUE_TYPE, PAIR_DIM),
                             jnp.float32),
           pltpu.VMEM_SHARED((N_NODES,), jnp.int32)]
    )
    fn = pl.kernel(
        _body,
        out_type=jax.ShapeDtypeStruct((N_EDGES, PAIR_DIM), jnp.float32),
        mesh=mesh,
        scratch_types=scratch,
    )
    return fn(src, dst, residue.astype(jnp.int32), weight)
